# Initial kernel scaffold; baseline (speedup 1.0000x reference)
import functools

import jax
import jax.numpy as jnp
from jax import lax
from jax.experimental import pallas as pl
from jax.experimental.pallas import tpu as pltpu
from jax.experimental.pallas import tpu_sc as plsc

TT = 32768
DD = 128
NSEG = 16
LANES = 16
NVEC = DD // LANES
NC = 2
NS = 16
NW = NC * NS
RPW = TT // NW
C1 = 256
C3 = 256
CU_PAD = 32


def _cu_scalars(cu_v):
    cu_vec = cu_v[pl.ds(0, LANES)]
    lane_ids = lax.iota(jnp.int32, LANES)
    cus = [jnp.int32(0)]
    for s in range(1, NSEG):
        cus.append(jnp.sum(jnp.where(lane_ids == s, cu_vec, 0)))
    cus.append(jnp.int32(TT))
    return cus


def _pass1_body(flat_hbm, cu_hbm, part_hbm, buf, acc, cu_v):
    wid = lax.axis_index("c") * NS + lax.axis_index("s")
    base = wid * RPW
    pltpu.sync_copy(cu_hbm, cu_v)
    cus = _cu_scalars(cu_v)
    zero = jnp.zeros((LANES,), jnp.float32)
    for s in range(NSEG):
        for j in range(NVEC):
            acc[s, pl.ds(j * LANES, LANES)] = zero
    for k in range(RPW // C1):
        chunk_lo = base + k * C1
        pltpu.sync_copy(flat_hbm.at[pl.ds(chunk_lo, C1)], buf)
        for s in range(NSEG):
            lo = jnp.clip(cus[s], chunk_lo, chunk_lo + C1) - chunk_lo
            hi = jnp.clip(cus[s + 1], chunk_lo, chunk_lo + C1) - chunk_lo

            def body(i, carry):
                return tuple(carry[j] + buf[i, pl.ds(j * LANES, LANES)]
                             for j in range(NVEC))

            sums = lax.fori_loop(lo, hi, body,
                                 tuple(zero for _ in range(NVEC)))
            for j in range(NVEC):
                sl = pl.ds(j * LANES, LANES)
                acc[s, sl] = acc[s, sl] + sums[j]
    pltpu.sync_copy(acc, part_hbm.at[wid])


def _gate_body(cu_smem, part_ref, w_ref, gate_ref):
    psum = jnp.sum(part_ref[...], axis=0)
    m = jnp.dot(psum, w_ref[...], preferred_element_type=jnp.float32)
    rows = lax.broadcasted_iota(jnp.int32, (NSEG, DD), 0)
    inv = jnp.full((NSEG, DD), 1.0, dtype=jnp.float32)
    for s in range(NSEG):
        cnt = jnp.maximum(cu_smem[s + 1] - cu_smem[s], 1).astype(jnp.float32)
        inv = jnp.where(rows == s, 1.0 / cnt, inv)
    gate_ref[...] = jax.nn.sigmoid(m * inv)


def _pass3_body(flat_hbm, cu_hbm, gate_hbm, out_hbm, ibuf, obuf, gate_v, cu_v):
    wid = lax.axis_index("c") * NS + lax.axis_index("s")
    base = wid * RPW
    pltpu.sync_copy(cu_hbm, cu_v)
    pltpu.sync_copy(gate_hbm, gate_v)
    cus = _cu_scalars(cu_v)
    for k in range(RPW // C3):
        chunk_lo = base + k * C3
        pltpu.sync_copy(flat_hbm.at[pl.ds(chunk_lo, C3)], ibuf)
        for s in range(NSEG):
            lo = jnp.clip(cus[s], chunk_lo, chunk_lo + C3) - chunk_lo
            hi = jnp.clip(cus[s + 1], chunk_lo, chunk_lo + C3) - chunk_lo
            gvecs = [gate_v[s, pl.ds(j * LANES, LANES)] for j in range(NVEC)]

            def body(i, carry):
                for j in range(NVEC):
                    sl = pl.ds(j * LANES, LANES)
                    obuf[i, sl] = ibuf[i, sl] * gvecs[j]
                return carry

            lax.fori_loop(lo, hi, body, jnp.int32(0))
        pltpu.sync_copy(obuf, out_hbm.at[pl.ds(chunk_lo, C3)])


def kernel(flat, cu_seqlens, W):
    cu_pad = jnp.concatenate([
        cu_seqlens.astype(jnp.int32),
        jnp.full((CU_PAD - NSEG - 1,), TT, dtype=jnp.int32),
    ])

    mesh1 = plsc.VectorSubcoreMesh(core_axis_name="c", subcore_axis_name="s")
    pass1 = functools.partial(
        pl.kernel,
        out_type=jax.ShapeDtypeStruct((NW, NSEG, DD), jnp.float32),
        mesh=mesh1,
        scratch_types=[
            pltpu.VMEM((C1, DD), jnp.float32),
            pltpu.VMEM((NSEG, DD), jnp.float32),
            pltpu.VMEM((CU_PAD,), jnp.int32),
        ],
    )(_pass1_body)
    partials = pass1(flat, cu_pad)

    gate = pl.pallas_call(
        _gate_body,
        out_shape=jax.ShapeDtypeStruct((NSEG, DD), jnp.float32),
        in_specs=[
            pl.BlockSpec(memory_space=pltpu.SMEM),
            pl.BlockSpec(memory_space=pltpu.VMEM),
            pl.BlockSpec(memory_space=pltpu.VMEM),
        ],
    )(cu_pad, partials, W)

    mesh3 = plsc.VectorSubcoreMesh(core_axis_name="c", subcore_axis_name="s")
    pass3 = functools.partial(
        pl.kernel,
        out_type=jax.ShapeDtypeStruct((TT, DD), jnp.float32),
        mesh=mesh3,
        scratch_types=[
            pltpu.VMEM((C3, DD), jnp.float32),
            pltpu.VMEM((C3, DD), jnp.float32),
            pltpu.VMEM((NSEG, DD), jnp.float32),
            pltpu.VMEM((CU_PAD,), jnp.int32),
        ],
    )(_pass3_body)
    return pass3(flat, cu_pad, gate)


# speedup vs baseline: 3.0145x; 3.0145x over previous
import dataclasses
import functools

import jax
import jax.numpy as jnp
from jax import lax
from jax.experimental import pallas as pl
from jax.experimental.pallas import tpu as pltpu
from jax.experimental.pallas import tpu_sc as plsc

TT = 32768
DD = 128
NSEG = 16
LANES = 16
NVEC = DD // LANES
NC = 2
NS = 16
NW = NC * NS
RPW = TT // NW
C1 = 256
C3 = 256
CU_PAD = 32


def _sc_compiler_params():
    cp = pltpu.CompilerParams()
    if "needs_layout_passes" in pltpu.CompilerParams.__dataclass_fields__:
        cp = dataclasses.replace(cp, needs_layout_passes=False)
    return cp


def _cu_scalars(cu_v):
    cu_vec = cu_v[pl.ds(0, LANES)]
    lane_ids = lax.iota(jnp.int32, LANES)
    cus = [jnp.int32(0)]
    for s in range(1, NSEG):
        cus.append(jnp.sum(jnp.where(lane_ids == s, cu_vec, 0)))
    cus.append(jnp.int32(TT))
    return cus


def _pass1_body(flat_hbm, cu_hbm, part_hbm, buf, acc, cu_v):
    wid = lax.axis_index("c") * NS + lax.axis_index("s")
    base = wid * RPW
    pltpu.sync_copy(cu_hbm, cu_v)
    cus = _cu_scalars(cu_v)
    zero = jnp.zeros((LANES,), jnp.float32)
    for s in range(NSEG):
        for j in range(NVEC):
            acc[s, pl.ds(j * LANES, LANES)] = zero
    for k in range(RPW // C1):
        chunk_lo = base + k * C1
        pltpu.sync_copy(flat_hbm.at[pl.ds(chunk_lo, C1)], buf)
        for s in range(NSEG):
            lo = jnp.clip(cus[s], chunk_lo, chunk_lo + C1) - chunk_lo
            hi = jnp.clip(cus[s + 1], chunk_lo, chunk_lo + C1) - chunk_lo

            def body(i, carry):
                return tuple(carry[j] + buf[i, pl.ds(j * LANES, LANES)]
                             for j in range(NVEC))

            sums = lax.fori_loop(lo, hi, body,
                                 tuple(zero for _ in range(NVEC)))
            for j in range(NVEC):
                sl = pl.ds(j * LANES, LANES)
                acc[s, sl] = acc[s, sl] + sums[j]
    pltpu.sync_copy(acc, part_hbm.at[wid])


def _gate_body(cu_smem, part_ref, w_ref, gate_ref):
    psum = jnp.sum(part_ref[...], axis=0)
    m = jnp.dot(psum, w_ref[...], preferred_element_type=jnp.float32)
    rows = lax.broadcasted_iota(jnp.int32, (NSEG, DD), 0)
    inv = jnp.full((NSEG, DD), 1.0, dtype=jnp.float32)
    for s in range(NSEG):
        cnt = jnp.maximum(cu_smem[s + 1] - cu_smem[s], 1).astype(jnp.float32)
        inv = jnp.where(rows == s, 1.0 / cnt, inv)
    gate_ref[...] = jax.nn.sigmoid(m * inv)


def _pass3_body(flat_hbm, cu_hbm, gate_hbm, out_hbm, ibuf, obuf, gate_v, cu_v):
    wid = lax.axis_index("c") * NS + lax.axis_index("s")
    base = wid * RPW
    pltpu.sync_copy(cu_hbm, cu_v)
    pltpu.sync_copy(gate_hbm, gate_v)
    cus = _cu_scalars(cu_v)
    for k in range(RPW // C3):
        chunk_lo = base + k * C3
        pltpu.sync_copy(flat_hbm.at[pl.ds(chunk_lo, C3)], ibuf)
        for s in range(NSEG):
            lo = jnp.clip(cus[s], chunk_lo, chunk_lo + C3) - chunk_lo
            hi = jnp.clip(cus[s + 1], chunk_lo, chunk_lo + C3) - chunk_lo
            gvecs = [gate_v[s, pl.ds(j * LANES, LANES)] for j in range(NVEC)]

            def body(i, carry):
                for j in range(NVEC):
                    sl = pl.ds(j * LANES, LANES)
                    obuf[i, sl] = ibuf[i, sl] * gvecs[j]
                return carry

            lax.fori_loop(lo, hi, body, jnp.int32(0))
        pltpu.sync_copy(obuf, out_hbm.at[pl.ds(chunk_lo, C3)])


def kernel(flat, cu_seqlens, W):
    cu_pad = jnp.concatenate([
        cu_seqlens.astype(jnp.int32),
        jnp.full((CU_PAD - NSEG - 1,), TT, dtype=jnp.int32),
    ])

    mesh1 = plsc.VectorSubcoreMesh(core_axis_name="c", subcore_axis_name="s")
    pass1 = functools.partial(
        pl.kernel,
        out_type=jax.ShapeDtypeStruct((NW, NSEG, DD), jnp.float32),
        mesh=mesh1,
        compiler_params=_sc_compiler_params(),
        scratch_types=[
            pltpu.VMEM((C1, DD), jnp.float32),
            pltpu.VMEM((NSEG, DD), jnp.float32),
            pltpu.VMEM((CU_PAD,), jnp.int32),
        ],
    )(_pass1_body)
    partials = pass1(flat, cu_pad)

    gate = pl.pallas_call(
        _gate_body,
        out_shape=jax.ShapeDtypeStruct((NSEG, DD), jnp.float32),
        in_specs=[
            pl.BlockSpec(memory_space=pltpu.SMEM),
            pl.BlockSpec(memory_space=pltpu.VMEM),
            pl.BlockSpec(memory_space=pltpu.VMEM),
        ],
    )(cu_pad, partials, W)

    mesh3 = plsc.VectorSubcoreMesh(core_axis_name="c", subcore_axis_name="s")
    pass3 = functools.partial(
        pl.kernel,
        out_type=jax.ShapeDtypeStruct((TT, DD), jnp.float32),
        mesh=mesh3,
        compiler_params=_sc_compiler_params(),
        scratch_types=[
            pltpu.VMEM((C3, DD), jnp.float32),
            pltpu.VMEM((C3, DD), jnp.float32),
            pltpu.VMEM((NSEG, DD), jnp.float32),
            pltpu.VMEM((CU_PAD,), jnp.int32),
        ],
    )(_pass3_body)
    return pass3(flat, cu_pad, gate)


# speedup vs baseline: 3.8677x; 1.2830x over previous
import dataclasses
import functools

import jax
import jax.numpy as jnp
from jax import lax
from jax.experimental import pallas as pl
from jax.experimental.pallas import tpu as pltpu
from jax.experimental.pallas import tpu_sc as plsc

TT = 32768
DD = 128
NSEG = 16
LANES = 16
NVEC = DD // LANES
NC = 2
NS = 16
NW = NC * NS
RPW = TT // NW
C1 = 256
C3 = 128
CU_PAD = 32


def _sc_compiler_params():
    cp = pltpu.CompilerParams()
    if "needs_layout_passes" in pltpu.CompilerParams.__dataclass_fields__:
        cp = dataclasses.replace(cp, needs_layout_passes=False)
    return cp


def _cu_scalars(cu_v):
    cu_vec = cu_v[pl.ds(0, LANES)]
    lane_ids = lax.iota(jnp.int32, LANES)
    cus = [jnp.int32(0)]
    for s in range(1, NSEG):
        cus.append(jnp.sum(jnp.where(lane_ids == s, cu_vec, 0)))
    cus.append(jnp.int32(TT))
    return cus


def _pass1_body(flat_hbm, cu_hbm, part_hbm, buf, acc, cu_v, isem0, isem1):
    isems = (isem0, isem1)
    wid = lax.axis_index("c") * NS + lax.axis_index("s")
    base = wid * RPW
    pltpu.sync_copy(cu_hbm, cu_v)
    cus = _cu_scalars(cu_v)
    zero = jnp.zeros((LANES,), jnp.float32)
    for s in range(NSEG):
        for j in range(NVEC):
            acc[s, pl.ds(j * LANES, LANES)] = zero
    nch = RPW // C1
    pltpu.async_copy(flat_hbm.at[pl.ds(base, C1)], buf.at[0], isems[0])
    for k in range(nch):
        b = k % 2
        chunk_lo = base + k * C1
        if k + 1 < nch:
            nb = (k + 1) % 2
            pltpu.async_copy(flat_hbm.at[pl.ds(chunk_lo + C1, C1)],
                             buf.at[nb], isems[nb])
        pltpu.make_async_copy(flat_hbm.at[pl.ds(chunk_lo, C1)],
                              buf.at[b], isems[b]).wait()
        for s in range(NSEG):
            lo = jnp.clip(cus[s], chunk_lo, chunk_lo + C1) - chunk_lo
            hi = jnp.clip(cus[s + 1], chunk_lo, chunk_lo + C1) - chunk_lo

            @pl.when(hi > lo)
            def _run(b=b, s=s, lo=lo, hi=hi):
                def row(i, c):
                    return tuple(c[j] + buf[b, i, pl.ds(j * LANES, LANES)]
                                 for j in range(NVEC))

                sums = plsc.parallel_loop(
                    lo, hi, unroll=2,
                    carry=tuple(zero for _ in range(NVEC)))(row)
                for j in range(NVEC):
                    sl = pl.ds(j * LANES, LANES)
                    acc[s, sl] = acc[s, sl] + sums[j]
    pltpu.sync_copy(acc, part_hbm.at[wid])


def _gate_body(cu_smem, part_ref, w_ref, gate_ref):
    psum = jnp.sum(part_ref[...], axis=0)
    m = jnp.dot(psum, w_ref[...], preferred_element_type=jnp.float32)
    rows = lax.broadcasted_iota(jnp.int32, (NSEG, DD), 0)
    inv = jnp.full((NSEG, DD), 1.0, dtype=jnp.float32)
    for s in range(NSEG):
        cnt = jnp.maximum(cu_smem[s + 1] - cu_smem[s], 1).astype(jnp.float32)
        inv = jnp.where(rows == s, 1.0 / cnt, inv)
    gate_ref[...] = jax.nn.sigmoid(m * inv)


def _pass3_body(flat_hbm, cu_hbm, gate_hbm, out_hbm,
                ibuf, obuf, gate_v, cu_v, isem0, isem1, osem0, osem1):
    isems = (isem0, isem1)
    osems = (osem0, osem1)
    wid = lax.axis_index("c") * NS + lax.axis_index("s")
    base = wid * RPW
    pltpu.sync_copy(cu_hbm, cu_v)
    pltpu.sync_copy(gate_hbm, gate_v)
    cus = _cu_scalars(cu_v)
    nch = RPW // C3
    pltpu.async_copy(flat_hbm.at[pl.ds(base, C3)], ibuf.at[0], isems[0])
    pltpu.async_copy(flat_hbm.at[pl.ds(base + C3, C3)], ibuf.at[1], isems[1])

    @pl.loop(0, nch, step=2)
    def _chunks(k):
        for b in range(2):
            kb = k + b
            chunk_lo = base + kb * C3
            pltpu.make_async_copy(flat_hbm.at[pl.ds(chunk_lo, C3)],
                                  ibuf.at[b], isems[b]).wait()

            @pl.when(kb >= 2)
            def _wait_out(b=b, chunk_lo=chunk_lo):
                pltpu.make_async_copy(
                    obuf.at[b], out_hbm.at[pl.ds(chunk_lo - 2 * C3, C3)],
                    osems[b]).wait()

            for s in range(NSEG):
                lo = jnp.clip(cus[s], chunk_lo, chunk_lo + C3) - chunk_lo
                hi = jnp.clip(cus[s + 1], chunk_lo, chunk_lo + C3) - chunk_lo

                @pl.when(hi > lo)
                def _run(b=b, s=s, lo=lo, hi=hi):
                    gvecs = [gate_v[s, pl.ds(j * LANES, LANES)]
                             for j in range(NVEC)]

                    def row(i):
                        for j in range(NVEC):
                            sl = pl.ds(j * LANES, LANES)
                            obuf[b, i, sl] = ibuf[b, i, sl] * gvecs[j]

                    plsc.parallel_loop(lo, hi, unroll=2)(row)

            pltpu.async_copy(obuf.at[b], out_hbm.at[pl.ds(chunk_lo, C3)],
                             osems[b])

            @pl.when(kb + 2 < nch)
            def _next_in(b=b, chunk_lo=chunk_lo):
                pltpu.async_copy(flat_hbm.at[pl.ds(chunk_lo + 2 * C3, C3)],
                                 ibuf.at[b], isems[b])

    for b in range(2):
        pltpu.make_async_copy(
            obuf.at[b], out_hbm.at[pl.ds(base + (nch - 2 + b) * C3, C3)],
            osems[b]).wait()


def kernel(flat, cu_seqlens, W):
    cu_pad = jnp.concatenate([
        cu_seqlens.astype(jnp.int32),
        jnp.full((CU_PAD - NSEG - 1,), TT, dtype=jnp.int32),
    ])

    mesh1 = plsc.VectorSubcoreMesh(core_axis_name="c", subcore_axis_name="s")
    pass1 = functools.partial(
        pl.kernel,
        out_type=jax.ShapeDtypeStruct((NW, NSEG, DD), jnp.float32),
        mesh=mesh1,
        compiler_params=_sc_compiler_params(),
        scratch_types=[
            pltpu.VMEM((2, C1, DD), jnp.float32),
            pltpu.VMEM((NSEG, DD), jnp.float32),
            pltpu.VMEM((CU_PAD,), jnp.int32),
            pltpu.SemaphoreType.DMA,
            pltpu.SemaphoreType.DMA,
        ],
    )(_pass1_body)
    partials = pass1(flat, cu_pad)

    gate = pl.pallas_call(
        _gate_body,
        out_shape=jax.ShapeDtypeStruct((NSEG, DD), jnp.float32),
        in_specs=[
            pl.BlockSpec(memory_space=pltpu.SMEM),
            pl.BlockSpec(memory_space=pltpu.VMEM),
            pl.BlockSpec(memory_space=pltpu.VMEM),
        ],
    )(cu_pad, partials, W)

    mesh3 = plsc.VectorSubcoreMesh(core_axis_name="c", subcore_axis_name="s")
    pass3 = functools.partial(
        pl.kernel,
        out_type=jax.ShapeDtypeStruct((TT, DD), jnp.float32),
        mesh=mesh3,
        compiler_params=_sc_compiler_params(),
        scratch_types=[
            pltpu.VMEM((2, C3, DD), jnp.float32),
            pltpu.VMEM((2, C3, DD), jnp.float32),
            pltpu.VMEM((NSEG, DD), jnp.float32),
            pltpu.VMEM((CU_PAD,), jnp.int32),
            pltpu.SemaphoreType.DMA,
            pltpu.SemaphoreType.DMA,
            pltpu.SemaphoreType.DMA,
            pltpu.SemaphoreType.DMA,
        ],
    )(_pass3_body)
    return pass3(flat, cu_pad, gate)
